# baseline (device time: 28103 ns/iter reference)
import jax
import jax.numpy as jnp
from jax import lax
from jax.experimental import pallas as pl
from jax.experimental.pallas import tpu as pltpu

N_DEV = 16
N_Z = 4
N_S = 4


def kernel(q, k, v):
    s_per, d = q.shape
    scale = 1.0 / (d**0.5)

    def body(
        q_ref,
        k_ref,
        v_ref,
        out_ref,
        col_buf,
        plane_buf,
        colsend_sems,
        colrecv_sems,
        psend_sems,
        precv_sems,
    ):
        my = lax.axis_index("i")
        z = my // N_S
        s = lax.rem(my, N_S)

        def col_peer(dz):
            return lax.rem(z + dz, N_Z) * N_S + s

        def plane_peer(ds):
            return z * N_S + lax.rem(s + ds, N_S)

        col_buf[z, 0] = k_ref[...].astype(jnp.bfloat16)
        col_buf[z, 1] = v_ref[...].astype(jnp.bfloat16)

        barrier = pltpu.get_barrier_semaphore()
        for dz in range(1, N_Z):
            pl.semaphore_signal(
                barrier,
                inc=1,
                device_id=(col_peer(dz),),
                device_id_type=pl.DeviceIdType.MESH,
            )
        for ds in range(1, N_S):
            pl.semaphore_signal(
                barrier,
                inc=1,
                device_id=(plane_peer(ds),),
                device_id_type=pl.DeviceIdType.MESH,
            )
        pl.semaphore_wait(barrier, 6)

        sends = []

        for dz in range(1, N_Z):
            rdma = pltpu.make_async_remote_copy(
                src_ref=col_buf.at[z],
                dst_ref=col_buf.at[z],
                send_sem=colsend_sems.at[dz - 1],
                recv_sem=colrecv_sems.at[z],
                device_id=(col_peer(dz),),
                device_id_type=pl.DeviceIdType.MESH,
            )
            rdma.start()
            sends.append(rdma)

        q_val = (q_ref[...] * scale).astype(jnp.bfloat16)
        l = jnp.zeros((s_per, 1), dtype=jnp.float32)
        acc = jnp.zeros((s_per, d), dtype=jnp.float32)

        def accumulate(kj, vj, l, acc):
            sc = lax.dot_general(
                q_val,
                kj,
                (((1,), (1,)), ((), ())),
                preferred_element_type=jnp.float32,
            )
            p = jnp.exp(sc)
            l = l + jnp.sum(p, axis=1, keepdims=True)
            acc = acc + lax.dot(
                p.astype(jnp.bfloat16), vj, preferred_element_type=jnp.float32
            )
            return l, acc

        for dz in range(1, N_Z):
            zp = lax.rem(z + dz, N_Z)
            recv = pltpu.make_async_remote_copy(
                src_ref=col_buf.at[z],
                dst_ref=col_buf.at[zp],
                send_sem=colsend_sems.at[0],
                recv_sem=colrecv_sems.at[zp],
                device_id=(my,),
                device_id_type=pl.DeviceIdType.MESH,
            )
            recv.wait_recv()

        for ds in (2, 1, 3):
            rdma = pltpu.make_async_remote_copy(
                src_ref=col_buf,
                dst_ref=plane_buf.at[3 - ds],
                send_sem=psend_sems.at[ds - 1],
                recv_sem=precv_sems.at[3 - ds],
                device_id=(plane_peer(ds),),
                device_id_type=pl.DeviceIdType.MESH,
            )
            rdma.start()
            sends.append(rdma)

        l, acc = accumulate(col_buf[z, 0], col_buf[z, 1], l, acc)
        for dz in range(1, N_Z):
            zp = lax.rem(z + dz, N_Z)
            l, acc = accumulate(col_buf[zp, 0], col_buf[zp, 1], l, acc)

        for p in (0, 2, 1):
            recv = pltpu.make_async_remote_copy(
                src_ref=col_buf,
                dst_ref=plane_buf.at[p],
                send_sem=psend_sems.at[0],
                recv_sem=precv_sems.at[p],
                device_id=(my,),
                device_id_type=pl.DeviceIdType.MESH,
            )
            recv.wait_recv()
            for zi in range(N_Z):
                l, acc = accumulate(
                    plane_buf[p, zi, 0], plane_buf[p, zi, 1], l, acc
                )

        for rdma in sends:
            rdma.wait_send()

        out_ref[...] = acc / l

    return pl.pallas_call(
        body,
        out_shape=jax.ShapeDtypeStruct((s_per, d), jnp.float32),
        in_specs=[pl.BlockSpec(memory_space=pltpu.VMEM)] * 3,
        out_specs=pl.BlockSpec(memory_space=pltpu.VMEM),
        scratch_shapes=[
            pltpu.VMEM((N_Z, 2, s_per, d), jnp.bfloat16),
            pltpu.VMEM((N_S - 1, N_Z, 2, s_per, d), jnp.bfloat16),
            pltpu.SemaphoreType.DMA((N_Z - 1,)),
            pltpu.SemaphoreType.DMA((N_Z,)),
            pltpu.SemaphoreType.DMA((N_S - 1,)),
            pltpu.SemaphoreType.DMA((N_S - 1,)),
        ],
        compiler_params=pltpu.CompilerParams(collective_id=0),
    )(q, k, v)
